# baseline (speedup 1.0000x reference)
import functools
import math

import numpy as np
import jax
import jax.numpy as jnp
from jax import lax
from jax.experimental import pallas as pl
from jax.experimental.pallas import tpu as pltpu
from jax.experimental.pallas import tpu_sc as plsc

VOCAB = 1000000
DIM = 64
B = 4096
L = 200
SCALE = math.sqrt(DIM)

NW = 32
ROWS_W = (B * L) // NW
CHUNK = 128
NCHUNK = ROWS_W // CHUNK
NBUF = 4
AHEAD = 3
STREAM_ROWS = 128


def _make_pe2() -> np.ndarray:
    position = np.arange(0, L, dtype=np.float32)[:, None]
    div_term = np.exp(
        np.arange(0, DIM, 2, dtype=np.float32) * -(math.log(10000.0) / DIM))
    pe = np.zeros((L, DIM), dtype=np.float32)
    pe[:, 0::2] = np.sin(position * div_term)
    pe[:, 1::2] = np.cos(position * div_term)
    return np.concatenate([pe, pe], axis=0)


_PE2 = _make_pe2()

_mesh = plsc.VectorSubcoreMesh(core_axis_name="c", subcore_axis_name="s")


@functools.partial(
    pl.kernel,
    mesh=_mesh,
    out_type=jax.ShapeDtypeStruct((B * L, DIM), jnp.float32),
    compiler_params=pltpu.CompilerParams(
        use_tc_tiling_on_sc=False, needs_layout_passes=False),
    scratch_types=[
        pltpu.VMEM((NCHUNK, CHUNK), jnp.int32),
        pltpu.VMEM((2 * L, DIM), jnp.float32),
        pltpu.VMEM((NBUF, CHUNK, DIM), jnp.float32),
        pltpu.VMEM((NBUF, CHUNK, DIM), jnp.float32),
        pltpu.SemaphoreType.DMA((NBUF,)),
        pltpu.SemaphoreType.DMA((NBUF,)),
        pltpu.SemaphoreType.DMA((NBUF,)),
    ],
)
def _emb(x_hbm, pe_hbm, table_hbm, out_hbm,
         idx_v, pe_v, rows_v, outb_v, gsem, dsem, ssem):
    wid = lax.axis_index("s") * 2 + lax.axis_index("c")
    pltpu.sync_copy(x_hbm.at[wid], idx_v)
    pltpu.sync_copy(pe_hbm, pe_v)
    base = wid * ROWS_W

    def g_stream(c, b):
        return pltpu.make_async_copy(
            table_hbm.at[idx_v.at[c, pl.ds(0, STREAM_ROWS)]],
            rows_v.at[b, pl.ds(0, STREAM_ROWS)],
            gsem.at[b])

    def g_rows_issue(c, b):
        def blk(j16, carry):
            j0 = STREAM_ROWS + j16 * 16
            vidx = idx_v[c, pl.ds(j0, 16)]
            for r in range(16):
                pltpu.make_async_copy(
                    table_hbm.at[pl.ds(vidx[r], 1)],
                    rows_v.at[b, pl.ds(j0 + r, 1)],
                    dsem.at[b]).start()
            return carry
        lax.fori_loop(0, (CHUNK - STREAM_ROWS) // 16, blk, 0)

    def g_issue(c, b):
        g_stream(c, b).start()
        if STREAM_ROWS < CHUNK:
            g_rows_issue(c, b)

    def g_drain(c, b):
        g_stream(c, b).wait()
        if STREAM_ROWS < CHUNK:
            pltpu.make_async_copy(
                table_hbm.at[pl.ds(0, CHUNK - STREAM_ROWS)],
                rows_v.at[b, pl.ds(STREAM_ROWS, CHUNK - STREAM_ROWS)],
                dsem.at[b]).wait()

    def s_copy(c, b):
        return pltpu.make_async_copy(
            outb_v.at[b], out_hbm.at[pl.ds(base + c * CHUNK, CHUNK)],
            ssem.at[b])

    def compute(c, b):
        off = (c * CHUNK) % L

        @plsc.parallel_loop(0, CHUNK // 16)
        def blk_body(j16):
            vidx = idx_v[c, pl.ds(j16 * 16, 16)]
            nz = plsc.all_reduce_population_count(vidx == 0)

            @pl.when(nz[0] == 0)
            def _():
                for r in range(16):
                    j = j16 * 16 + r
                    for k in range(4):
                        sl = pl.ds(k * 16, 16)
                        outb_v[b, j, sl] = (
                            rows_v[b, j, sl] * jnp.float32(SCALE)
                            + pe_v[off + j, sl])

            @pl.when(nz[0] > 0)
            def _():
                fv = jnp.where(vidx != 0,
                               jnp.float32(SCALE), jnp.float32(0.0))
                for r in range(16):
                    j = j16 * 16 + r
                    fr = jnp.full((16,), fv[r], jnp.float32)
                    for k in range(4):
                        sl = pl.ds(k * 16, 16)
                        outb_v[b, j, sl] = (
                            rows_v[b, j, sl] * fr + pe_v[off + j, sl])

    for i in range(AHEAD):
        g_issue(i, i)

    def outer(g, carry):
        for bb in range(NBUF):
            c = g * NBUF + bb
            g_drain(c, bb)
            bn = (bb + AHEAD) % NBUF

            @pl.when(c + AHEAD < NCHUNK)
            def _():
                g_issue(c + AHEAD, bn)

            compute(c, bb)
            s_copy(c, bb).start()
            cd = c + AHEAD - NBUF

            @pl.when(cd >= 0)
            def _():
                s_copy(cd, bn).wait()

        return carry

    lax.fori_loop(0, NCHUNK // NBUF, outer, 0)
    for c in range(max(0, NCHUNK - NBUF + AHEAD), NCHUNK):
        s_copy(c, c % NBUF).wait()


def kernel(x, table):
    x3 = x.reshape(NW, NCHUNK, CHUNK)
    out = _emb(x3, _PE2, table)
    return out.reshape(B, L, DIM)


# speedup vs baseline: 1.1400x; 1.1400x over previous
import functools
import math

import numpy as np
import jax
import jax.numpy as jnp
from jax import lax
from jax.experimental import pallas as pl
from jax.experimental.pallas import tpu as pltpu
from jax.experimental.pallas import tpu_sc as plsc

VOCAB = 1000000
DIM = 64
B = 4096
L = 200
SCALE = math.sqrt(DIM)

NW = 32
ROWS_W = (B * L) // NW
CHUNK = 128
NCHUNK = ROWS_W // CHUNK
NBUF = 4
AHEAD = 3
STREAM_ROWS = 128


def _make_pe2() -> np.ndarray:
    position = np.arange(0, L, dtype=np.float32)[:, None]
    div_term = np.exp(
        np.arange(0, DIM, 2, dtype=np.float32) * -(math.log(10000.0) / DIM))
    pe = np.zeros((L, DIM), dtype=np.float32)
    pe[:, 0::2] = np.sin(position * div_term)
    pe[:, 1::2] = np.cos(position * div_term)
    return np.concatenate([pe, pe], axis=0)


_PE2 = _make_pe2()

_mesh = plsc.VectorSubcoreMesh(core_axis_name="c", subcore_axis_name="s")


@functools.partial(
    pl.kernel,
    mesh=_mesh,
    out_type=jax.ShapeDtypeStruct((B * L, DIM), jnp.float32),
    compiler_params=pltpu.CompilerParams(
        use_tc_tiling_on_sc=False, needs_layout_passes=False),
    scratch_types=[
        pltpu.VMEM((NCHUNK, CHUNK), jnp.int32),
        pltpu.VMEM((2 * L, DIM), jnp.float32),
        pltpu.VMEM((NBUF, CHUNK, DIM), jnp.float32),
        pltpu.VMEM((NBUF, CHUNK, DIM), jnp.float32),
        pltpu.SemaphoreType.DMA((NBUF,)),
        pltpu.SemaphoreType.DMA((NBUF,)),
        pltpu.SemaphoreType.DMA((NBUF,)),
    ],
)
def _emb(x_hbm, pe_hbm, table_hbm, out_hbm,
         idx_v, pe_v, rows_v, outb_v, gsem, dsem, ssem):
    wid = lax.axis_index("s") * 2 + lax.axis_index("c")
    pltpu.sync_copy(x_hbm.at[wid], idx_v)
    pltpu.sync_copy(pe_hbm, pe_v)
    base = wid * ROWS_W

    def g_stream(c, b):
        return pltpu.make_async_copy(
            table_hbm.at[idx_v.at[c, pl.ds(0, STREAM_ROWS)]],
            rows_v.at[b, pl.ds(0, STREAM_ROWS)],
            gsem.at[b])

    def g_rows_issue(c, b):
        def blk(j16, carry):
            j0 = STREAM_ROWS + j16 * 16
            vidx = idx_v[c, pl.ds(j0, 16)]
            for r in range(16):
                pltpu.make_async_copy(
                    table_hbm.at[pl.ds(vidx[r], 1)],
                    rows_v.at[b, pl.ds(j0 + r, 1)],
                    dsem.at[b]).start()
            return carry
        lax.fori_loop(0, (CHUNK - STREAM_ROWS) // 16, blk, 0)

    def g_issue(c, b):
        g_stream(c, b).start()
        if STREAM_ROWS < CHUNK:
            g_rows_issue(c, b)

    def g_drain(c, b):
        g_stream(c, b).wait()
        if STREAM_ROWS < CHUNK:
            pltpu.make_async_copy(
                table_hbm.at[pl.ds(0, CHUNK - STREAM_ROWS)],
                rows_v.at[b, pl.ds(STREAM_ROWS, CHUNK - STREAM_ROWS)],
                dsem.at[b]).wait()

    def s_copy(c, b):
        return pltpu.make_async_copy(
            outb_v.at[b], out_hbm.at[pl.ds(base + c * CHUNK, CHUNK)],
            ssem.at[b])

    def compute(c, b):
        off = (c * CHUNK) % L

        @plsc.parallel_loop(0, CHUNK // 16)
        def blk_body(j16):
            vidx = idx_v[c, pl.ds(j16 * 16, 16)]
            fv = jnp.where(vidx != 0, jnp.float32(SCALE), jnp.float32(0.0))
            for r in range(16):
                j = j16 * 16 + r
                fr = jnp.full((16,), fv[r], jnp.float32)
                for k in range(4):
                    sl = pl.ds(k * 16, 16)
                    outb_v[b, j, sl] = (
                        rows_v[b, j, sl] * fr + pe_v[off + j, sl])

    for i in range(AHEAD):
        g_issue(i, i)

    def outer(g, carry):
        for bb in range(NBUF):
            c = g * NBUF + bb
            g_drain(c, bb)
            bn = (bb + AHEAD) % NBUF

            @pl.when(c + AHEAD < NCHUNK)
            def _():
                g_issue(c + AHEAD, bn)

            compute(c, bb)
            s_copy(c, bb).start()
            cd = c + AHEAD - NBUF

            @pl.when(cd >= 0)
            def _():
                s_copy(cd, bn).wait()

        return carry

    lax.fori_loop(0, NCHUNK // NBUF, outer, 0)
    for c in range(max(0, NCHUNK - NBUF + AHEAD), NCHUNK):
        s_copy(c, c % NBUF).wait()


def kernel(x, table):
    x3 = x.reshape(NW, NCHUNK, CHUNK)
    out = _emb(x3, _PE2, table)
    return out.reshape(B, L, DIM)


# speedup vs baseline: 1.1493x; 1.0082x over previous
import functools
import math

import numpy as np
import jax
import jax.numpy as jnp
from jax import lax
from jax.experimental import pallas as pl
from jax.experimental.pallas import tpu as pltpu
from jax.experimental.pallas import tpu_sc as plsc

VOCAB = 1000000
DIM = 64
B = 4096
L = 200
SCALE = math.sqrt(DIM)

NW = 32
ROWS_W = (B * L) // NW
CHUNK = 128
NCHUNK = ROWS_W // CHUNK
NBUF = 4
AHEAD = 3


def _make_pe2() -> np.ndarray:
    position = np.arange(0, L, dtype=np.float32)[:, None]
    div_term = np.exp(
        np.arange(0, DIM, 2, dtype=np.float32) * -(math.log(10000.0) / DIM))
    pe = np.zeros((L, DIM), dtype=np.float32)
    pe[:, 0::2] = np.sin(position * div_term)
    pe[:, 1::2] = np.cos(position * div_term)
    return np.concatenate([pe, pe], axis=0)


_PE2 = _make_pe2()

_mesh = plsc.VectorSubcoreMesh(core_axis_name="c", subcore_axis_name="s")


@functools.partial(
    pl.kernel,
    mesh=_mesh,
    out_type=jax.ShapeDtypeStruct((B * L, DIM), jnp.float32),
    compiler_params=pltpu.CompilerParams(
        use_tc_tiling_on_sc=False, needs_layout_passes=False),
    scratch_types=[
        pltpu.VMEM((NCHUNK, CHUNK), jnp.int32),
        pltpu.VMEM((2 * L, DIM), jnp.float32),
        pltpu.VMEM((NBUF, CHUNK, DIM), jnp.float32),
        pltpu.VMEM((NBUF, CHUNK, DIM), jnp.float32),
        pltpu.SemaphoreType.DMA((NBUF,)),
        pltpu.SemaphoreType.DMA((NBUF,)),
    ],
)
def _emb(x_hbm, pe_hbm, table_hbm, out_hbm,
         idx_v, pe_v, rows_v, outb_v, gsem, ssem):
    wid = lax.axis_index("s") * 2 + lax.axis_index("c")
    pltpu.sync_copy(x_hbm.at[wid], idx_v)
    pltpu.sync_copy(pe_hbm, pe_v)
    base = wid * ROWS_W

    def g_copy(c, b):
        return pltpu.make_async_copy(
            table_hbm.at[idx_v.at[c]], rows_v.at[b], gsem.at[b])

    def s_copy(c, b):
        return pltpu.make_async_copy(
            outb_v.at[b], out_hbm.at[pl.ds(base + c * CHUNK, CHUNK)],
            ssem.at[b])

    def compute(c, b):
        off = (c * CHUNK) % L

        @plsc.parallel_loop(0, CHUNK // 16)
        def blk_body(j16):
            vidx = idx_v[c, pl.ds(j16 * 16, 16)]
            fv = jnp.where(vidx != 0, jnp.float32(SCALE), jnp.float32(0.0))
            for r in range(16):
                j = j16 * 16 + r
                fr = jnp.full((16,), fv[r], jnp.float32)
                for k in range(4):
                    sl = pl.ds(k * 16, 16)
                    outb_v[b, j, sl] = (
                        rows_v[b, j, sl] * fr + pe_v[off + j, sl])

    for i in range(AHEAD):
        g_copy(i, i).start()

    def outer(g, carry):
        for bb in range(NBUF):
            c = g * NBUF + bb
            g_copy(c, bb).wait()
            bn = (bb + AHEAD) % NBUF

            @pl.when(c + AHEAD < NCHUNK)
            def _():
                g_copy(c + AHEAD, bn).start()

            compute(c, bb)
            s_copy(c, bb).start()
            cd = c + AHEAD - NBUF

            @pl.when(cd >= 0)
            def _():
                s_copy(cd, bn).wait()

        return carry

    lax.fori_loop(0, NCHUNK // NBUF, outer, 0)
    for c in range(max(0, NCHUNK - NBUF + AHEAD), NCHUNK):
        s_copy(c, c % NBUF).wait()


def kernel(x, table):
    x3 = x.reshape(NW, NCHUNK, CHUNK)
    out = _emb(x3, _PE2, table)
    return out.reshape(B, L, DIM)
